# Initial kernel scaffold; baseline (speedup 1.0000x reference)
import functools

import jax
import jax.numpy as jnp
from jax import lax
from jax.experimental import pallas as pl
from jax.experimental.pallas import tpu as pltpu
from jax.experimental.pallas import tpu_sc as plsc

N = 10000
E = 320000
D = 128

NC = 2
NS = 16
NW = NC * NS
EPW = E // NW
K = 80
NCH = EPW // K
RPS = N // NS
ZR = 125
HW = 16

_mesh = plsc.VectorSubcoreMesh(
    core_axis_name="c", subcore_axis_name="s", num_cores=NC, num_subcores=NS
)


def _worker_id():
    return lax.axis_index("s") * NC + lax.axis_index("c")


@functools.partial(
    pl.kernel,
    out_type=jax.ShapeDtypeStruct((NC, N, HW), jnp.float32),
    mesh=_mesh,
    scratch_types=[
        pltpu.VMEM((NCH, K), jnp.int32),
        pltpu.VMEM((K, HW), jnp.float32),
        pltpu.VMEM((ZR, HW), jnp.float32),
        pltpu.VMEM_SHARED((N, HW), jnp.float32),
        pltpu.SemaphoreType.DMA,
    ],
)
def _hist_kernel(dst_hbm, out_hbm, dst_v, ones_v, zb_v, acc_sh, sem):
    cid = lax.axis_index("c")
    sid = lax.axis_index("s")
    wid = _worker_id()

    @pl.loop(0, K)
    def _(r):
        ones_v[r, :] = jnp.full((HW,), 1.0, jnp.float32)

    @pl.loop(0, ZR)
    def _(r):
        zb_v[r, :] = jnp.zeros((HW,), jnp.float32)

    @pl.loop(0, RPS // ZR)
    def _(j):
        pltpu.sync_copy(zb_v, acc_sh.at[pl.ds(sid * RPS + j * ZR, ZR)])

    pltpu.async_copy(dst_hbm.at[pl.ds(wid * NCH, NCH)], dst_v, sem).wait()
    plsc.subcore_barrier()

    @pl.loop(0, NCH)
    def _(i):
        pltpu.sync_copy(ones_v, acc_sh.at[dst_v.at[i]], add=True)

    plsc.subcore_barrier()
    pltpu.sync_copy(
        acc_sh.at[pl.ds(sid * RPS, RPS)],
        out_hbm.at[cid, pl.ds(sid * RPS, RPS)],
    )


@functools.partial(
    pl.kernel,
    out_type=jax.ShapeDtypeStruct((NC, N, D), jnp.float32),
    mesh=_mesh,
    scratch_types=[
        pltpu.VMEM((NCH, K), jnp.int32),
        pltpu.VMEM((NCH, K), jnp.int32),
        pltpu.VMEM((K, D), jnp.float32),
        pltpu.VMEM((ZR, D), jnp.float32),
        pltpu.VMEM_SHARED((N, D), jnp.float32),
        pltpu.SemaphoreType.DMA,
        pltpu.SemaphoreType.DMA,
    ],
)
def _agg_kernel(h_hbm, src_hbm, dst_hbm, out_hbm,
                src_v, dst_v, rows_v, zb_v, acc_sh, sem0, sem1):
    cid = lax.axis_index("c")
    sid = lax.axis_index("s")
    wid = _worker_id()

    @pl.loop(0, ZR)
    def _(r):
        @pl.loop(0, D // 16)
        def _(cc):
            zb_v[r, pl.ds(cc * 16, 16)] = jnp.zeros((16,), jnp.float32)

    @pl.loop(0, RPS // ZR)
    def _(j):
        pltpu.sync_copy(zb_v, acc_sh.at[pl.ds(sid * RPS + j * ZR, ZR)])

    pltpu.async_copy(src_hbm.at[pl.ds(wid * NCH, NCH)], src_v, sem0)
    pltpu.async_copy(dst_hbm.at[pl.ds(wid * NCH, NCH)], dst_v, sem1)
    pltpu.make_async_copy(src_hbm.at[pl.ds(wid * NCH, NCH)], src_v, sem0).wait()
    pltpu.make_async_copy(dst_hbm.at[pl.ds(wid * NCH, NCH)], dst_v, sem1).wait()
    plsc.subcore_barrier()

    @pl.loop(0, NCH)
    def _(i):
        pltpu.async_copy(h_hbm.at[src_v.at[i]], rows_v, sem0).wait()
        pltpu.sync_copy(rows_v, acc_sh.at[dst_v.at[i]], add=True)

    plsc.subcore_barrier()
    pltpu.sync_copy(
        acc_sh.at[pl.ds(sid * RPS, RPS)],
        out_hbm.at[cid, pl.ds(sid * RPS, RPS)],
    )


_RB = 400
_GRID = N // _RB

_row_spec = pl.BlockSpec((_RB, D), lambda i: (i, 0))
_hist_spec = pl.BlockSpec((_RB, HW), lambda i: (i, 0))
_w_spec = pl.BlockSpec((D, D), lambda i: (0, 0))
_b_spec = pl.BlockSpec((1, D), lambda i: (0, 0))


def _dinv_of(h0_ref, h1_ref):
    deg = h0_ref[:, 0] + h1_ref[:, 0] + 1.0
    return lax.rsqrt(deg)


def _prep_body(x_ref, w_ref, h0_ref, h1_ref, o_ref):
    dinv = _dinv_of(h0_ref, h1_ref)
    h = jnp.dot(x_ref[...], w_ref[...], preferred_element_type=jnp.float32)
    o_ref[...] = h * dinv[:, None]


_prep = pl.pallas_call(
    _prep_body,
    grid=(_GRID,),
    in_specs=[_row_spec, _w_spec, _hist_spec, _hist_spec],
    out_specs=_row_spec,
    out_shape=jax.ShapeDtypeStruct((N, D), jnp.float32),
)


def _mid_body(p0_ref, p1_ref, hp_ref, b_ref, w_ref, h0_ref, h1_ref, o_ref):
    dinv = _dinv_of(h0_ref, h1_ref)
    pre = dinv[:, None] * (p0_ref[...] + p1_ref[...] + hp_ref[...]) + b_ref[...]
    act = jnp.maximum(pre, 0.0)
    h = jnp.dot(act, w_ref[...], preferred_element_type=jnp.float32)
    o_ref[...] = h * dinv[:, None]


_mid = pl.pallas_call(
    _mid_body,
    grid=(_GRID,),
    in_specs=[_row_spec, _row_spec, _row_spec, _b_spec, _w_spec,
              _hist_spec, _hist_spec],
    out_specs=_row_spec,
    out_shape=jax.ShapeDtypeStruct((N, D), jnp.float32),
)


def _final_body(p0_ref, p1_ref, hp_ref, b_ref, h0_ref, h1_ref, o_ref):
    dinv = _dinv_of(h0_ref, h1_ref)
    o_ref[...] = (
        dinv[:, None] * (p0_ref[...] + p1_ref[...] + hp_ref[...]) + b_ref[...]
    )


_final = pl.pallas_call(
    _final_body,
    grid=(_GRID,),
    in_specs=[_row_spec, _row_spec, _row_spec, _b_spec, _hist_spec, _hist_spec],
    out_specs=_row_spec,
    out_shape=jax.ShapeDtypeStruct((N, D), jnp.float32),
)


def kernel(x, edge_index, W1, b1, W2, b2):
    src = edge_index[0].astype(jnp.int32).reshape(NW * NCH, K)
    dst = edge_index[1].astype(jnp.int32).reshape(NW * NCH, K)
    b1r = b1.reshape(1, D)
    b2r = b2.reshape(1, D)

    hist = _hist_kernel(dst)
    h0, h1 = hist[0], hist[1]

    h1p = _prep(x, W1, h0, h1)
    p = _agg_kernel(h1p, src, dst)
    h2p = _mid(p[0], p[1], h1p, b1r, W2, h0, h1)
    q = _agg_kernel(h2p, src, dst)
    out = _final(q[0], q[1], h2p, b2r, h0, h1)
    return out


# speedup vs baseline: 16.3903x; 16.3903x over previous
import functools

import jax
import jax.numpy as jnp
from jax import lax
from jax.experimental import pallas as pl
from jax.experimental.pallas import tpu as pltpu
from jax.experimental.pallas import tpu_sc as plsc

N = 10000
E = 320000
D = 128

NC = 2
NS = 16
NW = NC * NS
EPW = E // NW
K = 80
NCH = EPW // K
RPS = N // NS
ZR = 125
HW = 16

_mesh = plsc.VectorSubcoreMesh(
    core_axis_name="c", subcore_axis_name="s", num_cores=NC, num_subcores=NS
)


def _worker_id():
    return lax.axis_index("s") * NC + lax.axis_index("c")


@functools.partial(
    pl.kernel,
    out_type=jax.ShapeDtypeStruct((NC, NS, RPS, D), jnp.float32),
    mesh=_mesh,
    scratch_types=[
        pltpu.VMEM((NCH, K), jnp.int32),
        pltpu.VMEM((K, D), jnp.float32),
        pltpu.VMEM_SHARED((N, D), jnp.float32),
        pltpu.SemaphoreType.DMA,
    ],
)
def _hist_kernel(dst_hbm, out_hbm, dst_v, ones_v, acc_sh, sem):
    cid = lax.axis_index("c")
    sid = lax.axis_index("s")
    wid = _worker_id()

    @pl.loop(0, K)
    def _(r):
        @pl.loop(0, D // 16)
        def _(cc):
            ones_v[r, pl.ds(cc * 16, 16)] = jnp.zeros((16,), jnp.float32)

    @pl.loop(0, RPS // K)
    def _(j):
        pltpu.sync_copy(ones_v, acc_sh.at[pl.ds(sid * RPS + j * K, K)])

    pltpu.sync_copy(
        ones_v.at[pl.ds(0, RPS - (RPS // K) * K)],
        acc_sh.at[pl.ds(sid * RPS + (RPS // K) * K, RPS - (RPS // K) * K)],
    )

    @pl.loop(0, K)
    def _(r):
        @pl.loop(0, D // 16)
        def _(cc):
            ones_v[r, pl.ds(cc * 16, 16)] = jnp.full((16,), 1.0, jnp.float32)

    pltpu.async_copy(dst_hbm.at[wid], dst_v, sem).wait()
    plsc.subcore_barrier()

    @pl.loop(0, NCH)
    def _(i):
        pltpu.sync_copy(ones_v, acc_sh.at[dst_v.at[i]], add=True)

    plsc.subcore_barrier()
    pltpu.sync_copy(
        acc_sh.at[pl.ds(sid * RPS, RPS)],
        out_hbm.at[cid, sid],
    )


@functools.partial(
    pl.kernel,
    out_type=jax.ShapeDtypeStruct((NC, NS, RPS, D), jnp.float32),
    mesh=_mesh,
    scratch_types=[
        pltpu.VMEM((NCH, K), jnp.int32),
        pltpu.VMEM((NCH, K), jnp.int32),
        pltpu.VMEM((K, D), jnp.float32),
        pltpu.VMEM_SHARED((N, D), jnp.float32),
        pltpu.SemaphoreType.DMA,
        pltpu.SemaphoreType.DMA,
    ],
)
def _agg_kernel(h_hbm, src_hbm, dst_hbm, out_hbm,
                src_v, dst_v, rows_v, acc_sh, sem0, sem1):
    cid = lax.axis_index("c")
    sid = lax.axis_index("s")
    wid = _worker_id()

    @pl.loop(0, K)
    def _(r):
        @pl.loop(0, D // 16)
        def _(cc):
            rows_v[r, pl.ds(cc * 16, 16)] = jnp.zeros((16,), jnp.float32)

    @pl.loop(0, RPS // K)
    def _(j):
        pltpu.sync_copy(rows_v, acc_sh.at[pl.ds(sid * RPS + j * K, K)])

    pltpu.sync_copy(
        rows_v.at[pl.ds(0, RPS - (RPS // K) * K)],
        acc_sh.at[pl.ds(sid * RPS + (RPS // K) * K, RPS - (RPS // K) * K)],
    )

    pltpu.async_copy(src_hbm.at[wid], src_v, sem0)
    pltpu.async_copy(dst_hbm.at[wid], dst_v, sem1)
    pltpu.make_async_copy(src_hbm.at[wid], src_v, sem0).wait()
    pltpu.make_async_copy(dst_hbm.at[wid], dst_v, sem1).wait()
    plsc.subcore_barrier()

    @pl.loop(0, NCH)
    def _(i):
        pltpu.async_copy(h_hbm.at[src_v.at[i]], rows_v, sem0).wait()
        pltpu.sync_copy(rows_v, acc_sh.at[dst_v.at[i]], add=True)

    plsc.subcore_barrier()
    pltpu.sync_copy(
        acc_sh.at[pl.ds(sid * RPS, RPS)],
        out_hbm.at[cid, sid],
    )


_RB = 400
_GRID = N // _RB

_row_spec = pl.BlockSpec((_RB, D), lambda i: (i, 0))
_hist_spec = pl.BlockSpec((_RB, D), lambda i: (i, 0))
_w_spec = pl.BlockSpec((D, D), lambda i: (0, 0))
_b_spec = pl.BlockSpec((1, D), lambda i: (0, 0))


def _dinv_of(h0_ref, h1_ref):
    deg = h0_ref[:, 0] + h1_ref[:, 0] + 1.0
    return lax.rsqrt(deg)


def _prep_body(x_ref, w_ref, h0_ref, h1_ref, o_ref):
    dinv = _dinv_of(h0_ref, h1_ref)
    h = jnp.dot(x_ref[...], w_ref[...], preferred_element_type=jnp.float32)
    o_ref[...] = h * dinv[:, None]


_prep = pl.pallas_call(
    _prep_body,
    grid=(_GRID,),
    in_specs=[_row_spec, _w_spec, _hist_spec, _hist_spec],
    out_specs=_row_spec,
    out_shape=jax.ShapeDtypeStruct((N, D), jnp.float32),
)


def _mid_body(p0_ref, p1_ref, hp_ref, b_ref, w_ref, h0_ref, h1_ref, o_ref):
    dinv = _dinv_of(h0_ref, h1_ref)
    pre = dinv[:, None] * (p0_ref[...] + p1_ref[...] + hp_ref[...]) + b_ref[...]
    act = jnp.maximum(pre, 0.0)
    h = jnp.dot(act, w_ref[...], preferred_element_type=jnp.float32)
    o_ref[...] = h * dinv[:, None]


_mid = pl.pallas_call(
    _mid_body,
    grid=(_GRID,),
    in_specs=[_row_spec, _row_spec, _row_spec, _b_spec, _w_spec,
              _hist_spec, _hist_spec],
    out_specs=_row_spec,
    out_shape=jax.ShapeDtypeStruct((N, D), jnp.float32),
)


def _final_body(p0_ref, p1_ref, hp_ref, b_ref, h0_ref, h1_ref, o_ref):
    dinv = _dinv_of(h0_ref, h1_ref)
    o_ref[...] = (
        dinv[:, None] * (p0_ref[...] + p1_ref[...] + hp_ref[...]) + b_ref[...]
    )


_final = pl.pallas_call(
    _final_body,
    grid=(_GRID,),
    in_specs=[_row_spec, _row_spec, _row_spec, _b_spec, _hist_spec, _hist_spec],
    out_specs=_row_spec,
    out_shape=jax.ShapeDtypeStruct((N, D), jnp.float32),
)


def kernel(x, edge_index, W1, b1, W2, b2):
    src = edge_index[0].astype(jnp.int32).reshape(NW, NCH, K)
    dst = edge_index[1].astype(jnp.int32).reshape(NW, NCH, K)
    b1r = b1.reshape(1, D)
    b2r = b2.reshape(1, D)

    hist = _hist_kernel(dst)
    h0 = hist[0].reshape(N, D)
    h1 = hist[1].reshape(N, D)

    h1p = _prep(x, W1, h0, h1)
    p = _agg_kernel(h1p, src, dst)
    h2p = _mid(p[0].reshape(N, D), p[1].reshape(N, D), h1p, b1r, W2, h0, h1)
    q = _agg_kernel(h2p, src, dst)
    out = _final(q[0].reshape(N, D), q[1].reshape(N, D), h2p, b2r, h0, h1)
    return out
